# baseline (speedup 1.0000x reference)
import functools

import jax
import jax.numpy as jnp
from jax import lax
from jax.experimental import pallas as pl
from jax.experimental.pallas import tpu as pltpu
from jax.experimental.pallas import tpu_sc as plsc

B = 16384
RANK = 16

_info = plsc.get_sparse_core_info()
NC = _info.num_cores
NS = _info.num_subcores
L = _info.num_lanes
NW = NC * NS
BPW = B // NW
CHUNK = 128
NCHUNK = BPW // CHUNK
NBLK = BPW // L

_mesh = plsc.VectorSubcoreMesh(core_axis_name="c", subcore_axis_name="s")


@functools.partial(
    pl.kernel,
    mesh=_mesh,
    out_type=(
        jax.ShapeDtypeStruct((B,), jnp.float32),
        jax.ShapeDtypeStruct((B,), jnp.float32),
    ),
    scratch_types=[
        pltpu.VMEM((NCHUNK, CHUNK), jnp.int32),
        pltpu.VMEM((NCHUNK, CHUNK), jnp.int32),
        pltpu.VMEM((NCHUNK, CHUNK), jnp.int32),
        pltpu.VMEM((BPW, RANK), jnp.float32),
        pltpu.VMEM((BPW, RANK), jnp.float32),
        pltpu.VMEM((BPW, RANK), jnp.float32),
        pltpu.VMEM((BPW,), jnp.float32),
        pltpu.VMEM((BPW,), jnp.float32),
        pltpu.SemaphoreType.DMA,
    ],
    compiler_params=pltpu.CompilerParams(
        needs_layout_passes=False, use_tc_tiling_on_sc=False),
)
def _bpr_sc(uids_hbm, pids_hbm, nids_hbm, uemb_hbm, iemb_hbm,
            outp_hbm, outn_hbm,
            uidx_v, pidx_v, nidx_v, urows_v, prows_v, nrows_v,
            outp_v, outn_v, sem):
    wid = lax.axis_index("s") * NC + lax.axis_index("c")

    row0 = wid * NCHUNK
    pltpu.sync_copy(uids_hbm.at[pl.ds(row0, NCHUNK)], uidx_v)
    pltpu.sync_copy(pids_hbm.at[pl.ds(row0, NCHUNK)], pidx_v)
    pltpu.sync_copy(nids_hbm.at[pl.ds(row0, NCHUNK)], nidx_v)

    copies = []
    for j in range(NCHUNK):
        dst = pl.ds(j * CHUNK, CHUNK)
        copies.append(pltpu.async_copy(
            uemb_hbm.at[uidx_v.at[j]], urows_v.at[dst], sem))
        copies.append(pltpu.async_copy(
            iemb_hbm.at[pidx_v.at[j]], prows_v.at[dst], sem))
        copies.append(pltpu.async_copy(
            iemb_hbm.at[nidx_v.at[j]], nrows_v.at[dst], sem))
    for c in copies:
        c.wait()

    iota = lax.iota(jnp.int32, L)

    def blk_body(b, carry):
        base = b * L
        rows = base + iota
        accp = jnp.zeros((L,), jnp.float32)
        accn = jnp.zeros((L,), jnp.float32)
        for k in range(RANK):
            col = jnp.full((L,), k, jnp.int32)
            u = plsc.load_gather(urows_v, [rows, col])
            p = plsc.load_gather(prows_v, [rows, col])
            n = plsc.load_gather(nrows_v, [rows, col])
            accp = accp + u * p
            accn = accn + u * n
        outp_v[pl.ds(base, L)] = accp
        outn_v[pl.ds(base, L)] = accn
        return carry

    lax.fori_loop(0, NBLK, blk_body, 0)

    out0 = wid * BPW
    pltpu.sync_copy(outp_v, outp_hbm.at[pl.ds(out0, BPW)])
    pltpu.sync_copy(outn_v, outn_hbm.at[pl.ds(out0, BPW)])


def kernel(user_ids, pos_items, neg_items, user_emb, item_emb):
    uids = user_ids.astype(jnp.int32).reshape(B // CHUNK, CHUNK)
    pids = pos_items.astype(jnp.int32).reshape(B // CHUNK, CHUNK)
    nids = neg_items.astype(jnp.int32).reshape(B // CHUNK, CHUNK)
    return _bpr_sc(uids, pids, nids, user_emb, item_emb)
